# baseline (speedup 1.0000x reference)
import jax
import jax.numpy as jnp
from jax.experimental import pallas as pl

N_NODES = 10000
N_EDGES = 5000


def _smooth(X, nidx, eidx, dvis, deinv):
    Xs = X * dvis[:, None]
    Ef = jax.ops.segment_sum(Xs[nidx], eidx, num_segments=N_EDGES)
    Ef = Ef * deinv[:, None]
    Y = jax.ops.segment_sum(Ef[eidx], nidx, num_segments=N_NODES)
    return Y * dvis[:, None]


def _mlp_body(z_ref, fc1_w_ref, fc1_b_ref, ln_g_ref, ln_b_ref, fc2_w_ref, fc2_b_ref, o_ref):
    z = z_ref[0, :]
    z = z @ fc1_w_ref[...] + fc1_b_ref[...]
    mu = jnp.mean(z)
    var = jnp.mean((z - mu) ** 2)
    z = (z - mu) / jnp.sqrt(var + 1e-5) * ln_g_ref[...] + ln_b_ref[...]
    z = z * jax.nn.sigmoid(z)
    z = z @ fc2_w_ref[...] + fc2_b_ref[...]
    nrm = jnp.maximum(jnp.sqrt(jnp.sum(z * z)), 1e-12)
    o_ref[0, :] = z / nrm


def kernel(X, nidx0, eidx0, nidx1, eidx1, W00, b00, W01, b01, W10, b10, W11, b11, fc1_w, fc1_b, ln_g, ln_b, fc2_w, fc2_b):
    pooled = []
    for nidx, eidx, Wa, ba, Wb, bb in (
        (nidx0, eidx0, W00, b00, W01, b01),
        (nidx1, eidx1, W10, b10, W11, b11),
    ):
        dv = jnp.bincount(nidx, length=N_NODES).astype(jnp.float32)
        de = jnp.bincount(eidx, length=N_EDGES).astype(jnp.float32)
        dvis = jnp.where(dv > 0, dv ** -0.5, 0.0)
        deinv = jnp.where(de > 0, 1.0 / de, 0.0)
        h = jax.nn.relu(_smooth(X @ Wa + ba, nidx, eidx, dvis, deinv))
        h = jax.nn.relu(_smooth(h @ Wb + bb, nidx, eidx, dvis, deinv))
        pooled.append(h.mean(axis=0))
    z = jnp.concatenate(pooled, axis=-1)[None, :]
    out = pl.pallas_call(
        _mlp_body,
        out_shape=jax.ShapeDtypeStruct((1, 128), jnp.float32),
    )(z, fc1_w, fc1_b, ln_g, ln_b, fc2_w, fc2_b)
    return out[0]


# speedup vs baseline: 5.9164x; 5.9164x over previous
import jax
import jax.numpy as jnp
from jax import lax
from jax.experimental import pallas as pl
from jax.experimental.pallas import tpu as pltpu
from jax.experimental.pallas import tpu_sc as plsc

N_NODES = 10000
N_NODES_PAD = 10240
N_EDGES = 5000
N_EDGES_PAD = 5120
N_MEM = 320000
NSUB = 16
HALF = 32
CW = 16
CCH = 100
PER_TILE = N_MEM // NSUB
NCH = PER_TILE // CCH
K = 8
NGRP = NCH // K
NROW_T = N_NODES_PAD // NSUB
EROW_T = N_EDGES_PAD // NSUB
R_BLK = 1024
NRB = N_NODES_PAD // R_BLK


def _counts_body(NI, EI, zn, ze, ones_h, outn, oute,
                 nbuf, ebuf, ones_v, accn, acce):
    c = lax.axis_index("c")
    s = lax.axis_index("s")
    pltpu.sync_copy(zn.at[pl.ds(s * NROW_T, NROW_T)],
                    accn.at[pl.ds(s * NROW_T, NROW_T)])
    pltpu.sync_copy(ze.at[pl.ds(s * EROW_T, EROW_T)],
                    acce.at[pl.ds(s * EROW_T, EROW_T)])
    pltpu.sync_copy(NI.at[c, pl.ds(s * NCH, NCH)], nbuf)
    pltpu.sync_copy(EI.at[c, pl.ds(s * NCH, NCH)], ebuf)
    pltpu.sync_copy(ones_h, ones_v)
    plsc.subcore_barrier()

    def body(j, carry):
        pltpu.sync_copy(ones_v, accn.at[nbuf.at[j]], add=True)
        pltpu.sync_copy(ones_v, acce.at[ebuf.at[j]], add=True)
        return carry

    lax.fori_loop(0, NCH, body, 0)
    plsc.subcore_barrier()
    pltpu.sync_copy(accn.at[pl.ds(s * NROW_T, NROW_T)],
                    outn.at[c, pl.ds(s * NROW_T, NROW_T)])
    pltpu.sync_copy(acce.at[pl.ds(s * EROW_T, EROW_T)],
                    oute.at[c, pl.ds(s * EROW_T, EROW_T)])


def _sc_counts(nidx0, eidx0, nidx1, eidx1):
    NI = jnp.stack([nidx0.reshape(-1, CCH), nidx1.reshape(-1, CCH)])
    EI = jnp.stack([eidx0.reshape(-1, CCH), eidx1.reshape(-1, CCH)])
    zn = jnp.zeros((N_NODES_PAD, CW), jnp.float32)
    ze = jnp.zeros((N_EDGES_PAD, CW), jnp.float32)
    ones_h = jnp.ones((CCH, CW), jnp.float32)
    mesh = plsc.VectorSubcoreMesh(core_axis_name="c", subcore_axis_name="s")
    fn = pl.kernel(
        _counts_body,
        mesh=mesh,
        compiler_params=pltpu.CompilerParams(use_tc_tiling_on_sc=False),
        out_type=[
            jax.ShapeDtypeStruct((2, N_NODES_PAD, CW), jnp.float32),
            jax.ShapeDtypeStruct((2, N_EDGES_PAD, CW), jnp.float32),
        ],
        scratch_types=[
            pltpu.VMEM((NCH, CCH), jnp.int32),
            pltpu.VMEM((NCH, CCH), jnp.int32),
            pltpu.VMEM((CCH, CW), jnp.float32),
            pltpu.VMEM_SHARED((N_NODES_PAD, CW), jnp.float32),
            pltpu.VMEM_SHARED((N_EDGES_PAD, CW), jnp.float32),
        ],
    )
    return fn(NI, EI, zn, ze, ones_h)


def _smooth_body(T, NI, EI, dvp, dip, zE, zS, O,
                 rows, di_v, dv_v, fbuf, T_sh, E_sh, S_sh):
    c = lax.axis_index("c")
    s = lax.axis_index("s")
    pltpu.sync_copy(dip.at[pl.ds(s * EROW_T, EROW_T)], di_v)
    pltpu.sync_copy(dvp.at[pl.ds(s * NROW_T, NROW_T)], dv_v)
    pltpu.sync_copy(zE.at[pl.ds(s * EROW_T, EROW_T)],
                    E_sh.at[pl.ds(s * EROW_T, EROW_T)])
    pltpu.sync_copy(zS.at[pl.ds(s * NROW_T, NROW_T)],
                    S_sh.at[pl.ds(s * NROW_T, NROW_T)])

    pltpu.sync_copy(T.at[c, pl.ds(s * NROW_T, NROW_T)], fbuf)

    def abody(r, carry):
        idx = jnp.zeros((16,), jnp.int32) + r
        d = plsc.load_gather(dv_v, [idx])
        a0 = fbuf[r, pl.ds(0, 16)]
        fbuf[r, pl.ds(0, 16)] = a0 * d
        a1 = fbuf[r, pl.ds(16, 16)]
        fbuf[r, pl.ds(16, 16)] = a1 * d
        return carry

    lax.fori_loop(0, NROW_T, abody, 0)
    pltpu.sync_copy(fbuf, T_sh.at[pl.ds(s * NROW_T, NROW_T)])
    plsc.subcore_barrier()

    def pairs_pass(IDXG, IDXS, src, dst):
        def scoped(gbuf, sbuf):
            def body(g, carry):
                base = s * NCH + g * K
                pltpu.sync_copy(IDXG.at[pl.ds(base, K)], gbuf)
                pltpu.sync_copy(IDXS.at[pl.ds(base, K)], sbuf)
                for b in range(K):
                    pltpu.sync_copy(src.at[gbuf.at[b]],
                                    rows.at[pl.ds(b * CCH, CCH)])
                for b in range(K):
                    pltpu.sync_copy(rows.at[pl.ds(b * CCH, CCH)],
                                    dst.at[sbuf.at[b]], add=True)
                return carry

            lax.fori_loop(0, NGRP, body, 0)

        pl.run_scoped(scoped,
                      pltpu.VMEM((K, CCH), jnp.int32),
                      pltpu.VMEM((K, CCH), jnp.int32))

    pairs_pass(NI, EI, T_sh, E_sh)
    plsc.subcore_barrier()

    pltpu.sync_copy(E_sh.at[pl.ds(s * EROW_T, EROW_T)],
                    fbuf.at[pl.ds(0, EROW_T)])

    def cbody(r, carry):
        idx = jnp.zeros((16,), jnp.int32) + r
        d = plsc.load_gather(di_v, [idx])
        a0 = fbuf[r, pl.ds(0, 16)]
        fbuf[r, pl.ds(0, 16)] = a0 * d
        a1 = fbuf[r, pl.ds(16, 16)]
        fbuf[r, pl.ds(16, 16)] = a1 * d
        return carry

    lax.fori_loop(0, EROW_T, cbody, 0)
    pltpu.sync_copy(fbuf.at[pl.ds(0, EROW_T)],
                    E_sh.at[pl.ds(s * EROW_T, EROW_T)])
    plsc.subcore_barrier()

    pairs_pass(EI, NI, E_sh, S_sh)
    plsc.subcore_barrier()

    pltpu.sync_copy(S_sh.at[pl.ds(s * NROW_T, NROW_T)], fbuf)

    def fbody(r, carry):
        idx = jnp.zeros((16,), jnp.int32) + r
        d = plsc.load_gather(dv_v, [idx])
        a0 = fbuf[r, pl.ds(0, 16)]
        fbuf[r, pl.ds(0, 16)] = jnp.maximum(a0 * d, 0.0)
        a1 = fbuf[r, pl.ds(16, 16)]
        fbuf[r, pl.ds(16, 16)] = jnp.maximum(a1 * d, 0.0)
        return carry

    lax.fori_loop(0, NROW_T, fbody, 0)
    pltpu.sync_copy(fbuf, O.at[c, pl.ds(s * NROW_T, NROW_T)])


def _sc_smooth(Tsplit, NIr, EIr, dvp, dip):
    zE = jnp.zeros((N_EDGES_PAD, HALF), jnp.float32)
    zS = jnp.zeros((N_NODES_PAD, HALF), jnp.float32)
    mesh = plsc.VectorSubcoreMesh(core_axis_name="c", subcore_axis_name="s")
    fn = pl.kernel(
        _smooth_body,
        mesh=mesh,
        compiler_params=pltpu.CompilerParams(needs_layout_passes=False,
                                             use_tc_tiling_on_sc=False),
        out_type=jax.ShapeDtypeStruct((2, N_NODES_PAD, HALF), jnp.float32),
        scratch_types=[
            pltpu.VMEM((K * CCH, HALF), jnp.float32),
            pltpu.VMEM((EROW_T,), jnp.float32),
            pltpu.VMEM((NROW_T,), jnp.float32),
            pltpu.VMEM((NROW_T, HALF), jnp.float32),
            pltpu.VMEM_SHARED((N_NODES_PAD, HALF), jnp.float32),
            pltpu.VMEM_SHARED((N_EDGES_PAD, HALF), jnp.float32),
            pltpu.VMEM_SHARED((N_NODES_PAD, HALF), jnp.float32),
        ],
    )
    return fn(Tsplit, NIr, EIr, dvp, dip, zE, zS)


def _prep_body(x_ref, w_ref, b_ref, o_ref):
    x = x_ref[...]
    w = w_ref[0]
    bias = b_ref[0, 0]
    t = jnp.dot(x, w, preferred_element_type=jnp.float32) + bias[None, :]
    o_ref[0, 0] = t[:, 0:HALF]
    o_ref[0, 1] = t[:, HALF:2 * HALF]


def _tc_prep(Xp, Wstk, bstk):
    return pl.pallas_call(
        _prep_body,
        grid=(2, NRB),
        in_specs=[
            pl.BlockSpec((R_BLK, 128), lambda b, r: (r, 0)),
            pl.BlockSpec((1, 128, 64), lambda b, r: (b, 0, 0)),
            pl.BlockSpec((1, 1, 64), lambda b, r: (b, 0, 0)),
        ],
        out_specs=pl.BlockSpec((1, 2, R_BLK, HALF), lambda b, r: (b, 0, r, 0)),
        out_shape=jax.ShapeDtypeStruct((2, 2, N_NODES_PAD, HALF), jnp.float32),
    )(Xp, Wstk, bstk)


def _mid_body(o_ref, w_ref, b_ref, t_ref):
    o0 = o_ref[0]
    o1 = o_ref[1]
    w = w_ref[...]
    bias = b_ref[0]
    t = (jnp.dot(o0, w[0:HALF, :], preferred_element_type=jnp.float32)
         + jnp.dot(o1, w[HALF:2 * HALF, :], preferred_element_type=jnp.float32)
         + bias[None, :])
    t_ref[0] = t[:, 0:HALF]
    t_ref[1] = t[:, HALF:2 * HALF]


def _tc_mid(O_in, W, b2):
    return pl.pallas_call(
        _mid_body,
        grid=(NRB,),
        in_specs=[
            pl.BlockSpec((2, R_BLK, HALF), lambda r: (0, r, 0)),
            pl.BlockSpec((64, 64), lambda r: (0, 0)),
            pl.BlockSpec((1, 64), lambda r: (0, 0)),
        ],
        out_specs=pl.BlockSpec((2, R_BLK, HALF), lambda r: (0, r, 0)),
        out_shape=jax.ShapeDtypeStruct((2, N_NODES_PAD, HALF), jnp.float32),
    )(O_in, W, b2)


def _final_body(o0_ref, o1_ref, fc1_w_ref, fc1_b_ref, ln_g_ref, ln_b_ref,
                fc2_w_ref, fc2_b_ref, out_ref):
    scale = 1.0 / N_NODES
    parts = [
        jnp.sum(o0_ref[0], axis=0) * scale,
        jnp.sum(o0_ref[1], axis=0) * scale,
        jnp.sum(o1_ref[0], axis=0) * scale,
        jnp.sum(o1_ref[1], axis=0) * scale,
    ]
    z = jnp.concatenate(parts, axis=-1)
    z = z @ fc1_w_ref[...] + fc1_b_ref[...]
    mu = jnp.mean(z)
    var = jnp.mean((z - mu) ** 2)
    z = (z - mu) / jnp.sqrt(var + 1e-5) * ln_g_ref[...] + ln_b_ref[...]
    z = z * jax.nn.sigmoid(z)
    z = z @ fc2_w_ref[...] + fc2_b_ref[...]
    nrm = jnp.maximum(jnp.sqrt(jnp.sum(z * z)), 1e-12)
    out_ref[0, :] = z / nrm


def _tc_final(O0, O1, fc1_w, fc1_b, ln_g, ln_b, fc2_w, fc2_b):
    return pl.pallas_call(
        _final_body,
        out_shape=jax.ShapeDtypeStruct((1, 128), jnp.float32),
    )(O0, O1, fc1_w, fc1_b, ln_g, ln_b, fc2_w, fc2_b)


def kernel(X, nidx0, eidx0, nidx1, eidx1, W00, b00, W01, b01, W10, b10, W11, b11, fc1_w, fc1_b, ln_g, ln_b, fc2_w, fc2_b):
    nidx0 = nidx0.astype(jnp.int32)
    eidx0 = eidx0.astype(jnp.int32)
    nidx1 = nidx1.astype(jnp.int32)
    eidx1 = eidx1.astype(jnp.int32)
    outn, oute = _sc_counts(nidx0, eidx0, nidx1, eidx1)
    dv = outn[:, :N_NODES, 0]
    de = oute[:, :N_EDGES, 0]
    dvis = jnp.where(dv > 0, lax.rsqrt(dv), 0.0)
    deinv = jnp.where(de > 0, 1.0 / de, 0.0)
    dvp = jnp.pad(dvis, ((0, 0), (0, N_NODES_PAD - N_NODES)))
    dip = jnp.pad(deinv, ((0, 0), (0, N_EDGES_PAD - N_EDGES)))

    Xp = jnp.pad(X, ((0, N_NODES_PAD - N_NODES), (0, 0)))
    Wstk = jnp.stack([W00, W10])
    bstk = jnp.stack([b00, b10]).reshape(2, 1, 64)
    T1 = _tc_prep(Xp, Wstk, bstk)

    outs = []
    for bi, (nidx, eidx, Wb, bb) in enumerate((
        (nidx0, eidx0, W01, b01),
        (nidx1, eidx1, W11, b11),
    )):
        NIr = nidx.reshape(-1, CCH)
        EIr = eidx.reshape(-1, CCH)
        h1 = _sc_smooth(T1[bi], NIr, EIr, dvp[bi], dip[bi])
        T2 = _tc_mid(h1, Wb, bb.reshape(1, 64))
        h2 = _sc_smooth(T2, NIr, EIr, dvp[bi], dip[bi])
        outs.append(h2)

    out = _tc_final(outs[0], outs[1], fc1_w, fc1_b, ln_g, ln_b, fc2_w, fc2_b)
    return out[0]


# speedup vs baseline: 6.8537x; 1.1584x over previous
import jax
import jax.numpy as jnp
from jax import lax
from jax.experimental import pallas as pl
from jax.experimental.pallas import tpu as pltpu
from jax.experimental.pallas import tpu_sc as plsc

N_NODES = 10000
N_NODES_PAD = 10240
N_EDGES = 5000
N_EDGES_PAD = 5120
N_MEM = 320000
N_MEM_PAD = 327680
NSUB = 16
FEAT = 64
NROW_T = N_NODES_PAD // NSUB
EROW_T = N_EDGES_PAD // NSUB
SUBROW = 320

CW = 16
CCH = 100
NCH = (N_MEM // NSUB) // CCH

SCCH = 128
SNCH = (N_MEM_PAD // NSUB) // SCCH
K = 4
NGRP = SNCH // K

R_BLK = 1024
NRB = N_NODES_PAD // R_BLK


def _counts_body(NI, EI, zn, ze, ones_h, outn, oute,
                 nbuf, ebuf, ones_v, accn, acce):
    c = lax.axis_index("c")
    s = lax.axis_index("s")
    pltpu.sync_copy(zn.at[pl.ds(s * NROW_T, NROW_T)],
                    accn.at[pl.ds(s * NROW_T, NROW_T)])
    pltpu.sync_copy(ze.at[pl.ds(s * EROW_T, EROW_T)],
                    acce.at[pl.ds(s * EROW_T, EROW_T)])
    pltpu.sync_copy(NI.at[c, pl.ds(s * NCH, NCH)], nbuf)
    pltpu.sync_copy(EI.at[c, pl.ds(s * NCH, NCH)], ebuf)
    pltpu.sync_copy(ones_h, ones_v)
    plsc.subcore_barrier()

    def body(j, carry):
        pltpu.sync_copy(ones_v, accn.at[nbuf.at[j]], add=True)
        pltpu.sync_copy(ones_v, acce.at[ebuf.at[j]], add=True)
        return carry

    lax.fori_loop(0, NCH, body, 0)
    plsc.subcore_barrier()
    pltpu.sync_copy(accn.at[pl.ds(s * NROW_T, NROW_T)],
                    outn.at[c, pl.ds(s * NROW_T, NROW_T)])
    pltpu.sync_copy(acce.at[pl.ds(s * EROW_T, EROW_T)],
                    oute.at[c, pl.ds(s * EROW_T, EROW_T)])


def _sc_counts(nidx0, eidx0, nidx1, eidx1):
    NI = jnp.stack([nidx0.reshape(-1, CCH), nidx1.reshape(-1, CCH)])
    EI = jnp.stack([eidx0.reshape(-1, CCH), eidx1.reshape(-1, CCH)])
    zn = jnp.zeros((N_NODES_PAD, CW), jnp.float32)
    ze = jnp.zeros((N_EDGES_PAD, CW), jnp.float32)
    ones_h = jnp.ones((CCH, CW), jnp.float32)
    mesh = plsc.VectorSubcoreMesh(core_axis_name="c", subcore_axis_name="s")
    fn = pl.kernel(
        _counts_body,
        mesh=mesh,
        compiler_params=pltpu.CompilerParams(use_tc_tiling_on_sc=False),
        out_type=[
            jax.ShapeDtypeStruct((2, N_NODES_PAD, CW), jnp.float32),
            jax.ShapeDtypeStruct((2, N_EDGES_PAD, CW), jnp.float32),
        ],
        scratch_types=[
            pltpu.VMEM((NCH, CCH), jnp.int32),
            pltpu.VMEM((NCH, CCH), jnp.int32),
            pltpu.VMEM((CCH, CW), jnp.float32),
            pltpu.VMEM_SHARED((N_NODES_PAD, CW), jnp.float32),
            pltpu.VMEM_SHARED((N_EDGES_PAD, CW), jnp.float32),
        ],
    )
    return fn(NI, EI, zn, ze, ones_h)


def _smooth_body(T, NI, EI, dvp, dip, zE, zS, O,
                 rows, di_v, dv_v, buf, TS_sh, E_sh):
    c = lax.axis_index("c")
    s = lax.axis_index("s")
    pltpu.sync_copy(dip.at[c, pl.ds(s * EROW_T, EROW_T)], di_v)
    pltpu.sync_copy(dvp.at[c, pl.ds(s * NROW_T, NROW_T)], dv_v)
    pltpu.sync_copy(zE.at[pl.ds(s * EROW_T, EROW_T)],
                    E_sh.at[pl.ds(s * EROW_T, EROW_T)])

    for half in range(2):
        base = s * NROW_T + half * SUBROW
        pltpu.sync_copy(T.at[c, pl.ds(base, SUBROW)], buf)

        def abody(r, carry):
            idx = jnp.zeros((16,), jnp.int32) + (half * SUBROW + r)
            d = plsc.load_gather(dv_v, [idx])
            for q in range(4):
                a = buf[r, pl.ds(q * 16, 16)]
                buf[r, pl.ds(q * 16, 16)] = a * d
            return carry

        lax.fori_loop(0, SUBROW, abody, 0)
        pltpu.sync_copy(buf, TS_sh.at[pl.ds(base, SUBROW)])
    plsc.subcore_barrier()

    def pairs_pass(IDXG, IDXS, src, dst):
        def scoped(gbuf, sbuf):
            def body(g, carry):
                base = s * SNCH + g * K
                pltpu.sync_copy(IDXG.at[c, pl.ds(base, K)], gbuf)
                pltpu.sync_copy(IDXS.at[c, pl.ds(base, K)], sbuf)
                for b in range(K):
                    pltpu.sync_copy(src.at[gbuf.at[b]],
                                    rows.at[pl.ds(b * SCCH, SCCH)])
                for b in range(K):
                    pltpu.sync_copy(rows.at[pl.ds(b * SCCH, SCCH)],
                                    dst.at[sbuf.at[b]], add=True)
                return carry

            lax.fori_loop(0, NGRP, body, 0)

        pl.run_scoped(scoped,
                      pltpu.VMEM((K, SCCH), jnp.int32),
                      pltpu.VMEM((K, SCCH), jnp.int32))

    pairs_pass(NI, EI, TS_sh, E_sh)
    plsc.subcore_barrier()

    pltpu.sync_copy(E_sh.at[pl.ds(s * EROW_T, EROW_T)],
                    buf.at[pl.ds(0, EROW_T)])

    def cbody(r, carry):
        idx = jnp.zeros((16,), jnp.int32) + r
        d = plsc.load_gather(di_v, [idx])
        for q in range(4):
            a = buf[r, pl.ds(q * 16, 16)]
            buf[r, pl.ds(q * 16, 16)] = a * d
        return carry

    lax.fori_loop(0, EROW_T, cbody, 0)
    pltpu.sync_copy(buf.at[pl.ds(0, EROW_T)],
                    E_sh.at[pl.ds(s * EROW_T, EROW_T)])

    for half in range(2):
        base = s * NROW_T + half * SUBROW
        pltpu.sync_copy(zS.at[pl.ds(base, SUBROW)],
                        TS_sh.at[pl.ds(base, SUBROW)])
    plsc.subcore_barrier()

    pairs_pass(EI, NI, E_sh, TS_sh)
    plsc.subcore_barrier()

    for half in range(2):
        base = s * NROW_T + half * SUBROW
        pltpu.sync_copy(TS_sh.at[pl.ds(base, SUBROW)], buf)

        def fbody(r, carry):
            idx = jnp.zeros((16,), jnp.int32) + (half * SUBROW + r)
            d = plsc.load_gather(dv_v, [idx])
            for q in range(4):
                a = buf[r, pl.ds(q * 16, 16)]
                buf[r, pl.ds(q * 16, 16)] = jnp.maximum(a * d, 0.0)
            return carry

        lax.fori_loop(0, SUBROW, fbody, 0)
        pltpu.sync_copy(buf, O.at[c, pl.ds(base, SUBROW)])


def _sc_smooth(T, NI, EI, dvp, dip):
    zE = jnp.zeros((N_EDGES_PAD, FEAT), jnp.float32)
    zS = jnp.zeros((N_NODES_PAD, FEAT), jnp.float32)
    mesh = plsc.VectorSubcoreMesh(core_axis_name="c", subcore_axis_name="s")
    fn = pl.kernel(
        _smooth_body,
        mesh=mesh,
        compiler_params=pltpu.CompilerParams(needs_layout_passes=False,
                                             use_tc_tiling_on_sc=False),
        out_type=jax.ShapeDtypeStruct((2, N_NODES_PAD, FEAT), jnp.float32),
        scratch_types=[
            pltpu.VMEM((K * SCCH, FEAT), jnp.float32),
            pltpu.VMEM((EROW_T,), jnp.float32),
            pltpu.VMEM((NROW_T,), jnp.float32),
            pltpu.VMEM((SUBROW, FEAT), jnp.float32),
            pltpu.VMEM_SHARED((N_NODES_PAD, FEAT), jnp.float32),
            pltpu.VMEM_SHARED((N_EDGES_PAD, FEAT), jnp.float32),
        ],
    )
    return fn(T, NI, EI, dvp, dip, zE, zS)


def _prep_body(x_ref, w_ref, b_ref, o_ref):
    x = x_ref[...]
    w = w_ref[0]
    bias = b_ref[0, 0]
    o_ref[0] = jnp.dot(x, w, preferred_element_type=jnp.float32) + bias[None, :]


def _tc_prep(Xp, Wstk, bstk):
    return pl.pallas_call(
        _prep_body,
        grid=(2, NRB),
        in_specs=[
            pl.BlockSpec((R_BLK, 128), lambda b, r: (r, 0)),
            pl.BlockSpec((1, 128, FEAT), lambda b, r: (b, 0, 0)),
            pl.BlockSpec((1, 1, FEAT), lambda b, r: (b, 0, 0)),
        ],
        out_specs=pl.BlockSpec((1, R_BLK, FEAT), lambda b, r: (b, r, 0)),
        out_shape=jax.ShapeDtypeStruct((2, N_NODES_PAD, FEAT), jnp.float32),
    )(Xp, Wstk, bstk)


def _mid_body(o_ref, w_ref, b_ref, t_ref):
    o = o_ref[0]
    w = w_ref[0]
    bias = b_ref[0, 0]
    t_ref[0] = (jnp.dot(o, w, preferred_element_type=jnp.float32)
                + bias[None, :])


def _tc_mid(O_in, Wstk, bstk):
    return pl.pallas_call(
        _mid_body,
        grid=(2, NRB),
        in_specs=[
            pl.BlockSpec((1, R_BLK, FEAT), lambda b, r: (b, r, 0)),
            pl.BlockSpec((1, FEAT, FEAT), lambda b, r: (b, 0, 0)),
            pl.BlockSpec((1, 1, FEAT), lambda b, r: (b, 0, 0)),
        ],
        out_specs=pl.BlockSpec((1, R_BLK, FEAT), lambda b, r: (b, r, 0)),
        out_shape=jax.ShapeDtypeStruct((2, N_NODES_PAD, FEAT), jnp.float32),
    )(O_in, Wstk, bstk)


def _final_body(o_ref, fc1_w_ref, fc1_b_ref, ln_g_ref, ln_b_ref,
                fc2_w_ref, fc2_b_ref, out_ref):
    scale = 1.0 / N_NODES
    parts = [
        jnp.sum(o_ref[0], axis=0) * scale,
        jnp.sum(o_ref[1], axis=0) * scale,
    ]
    z = jnp.concatenate(parts, axis=-1)
    z = z @ fc1_w_ref[...] + fc1_b_ref[...]
    mu = jnp.mean(z)
    var = jnp.mean((z - mu) ** 2)
    z = (z - mu) / jnp.sqrt(var + 1e-5) * ln_g_ref[...] + ln_b_ref[...]
    z = z * jax.nn.sigmoid(z)
    z = z @ fc2_w_ref[...] + fc2_b_ref[...]
    nrm = jnp.maximum(jnp.sqrt(jnp.sum(z * z)), 1e-12)
    out_ref[0, :] = z / nrm


def _tc_final(O_l2, fc1_w, fc1_b, ln_g, ln_b, fc2_w, fc2_b):
    return pl.pallas_call(
        _final_body,
        out_shape=jax.ShapeDtypeStruct((1, 128), jnp.float32),
    )(O_l2, fc1_w, fc1_b, ln_g, ln_b, fc2_w, fc2_b)


def _pad_idx(idx, fill):
    return jnp.pad(idx, (0, N_MEM_PAD - N_MEM),
                   constant_values=fill).reshape(-1, SCCH)


def kernel(X, nidx0, eidx0, nidx1, eidx1, W00, b00, W01, b01, W10, b10, W11, b11, fc1_w, fc1_b, ln_g, ln_b, fc2_w, fc2_b):
    nidx0 = nidx0.astype(jnp.int32)
    eidx0 = eidx0.astype(jnp.int32)
    nidx1 = nidx1.astype(jnp.int32)
    eidx1 = eidx1.astype(jnp.int32)
    outn, oute = _sc_counts(nidx0, eidx0, nidx1, eidx1)
    dv = outn[:, :N_NODES, 0]
    de = oute[:, :N_EDGES, 0]
    dvis = jnp.where(dv > 0, lax.rsqrt(dv), 0.0)
    deinv = jnp.where(de > 0, 1.0 / de, 0.0)
    dvp = jnp.pad(dvis, ((0, 0), (0, N_NODES_PAD - N_NODES)))
    dip = jnp.pad(deinv, ((0, 0), (0, N_EDGES_PAD - N_EDGES)))

    NI = jnp.stack([_pad_idx(nidx0, N_NODES), _pad_idx(nidx1, N_NODES)])
    EI = jnp.stack([_pad_idx(eidx0, N_EDGES), _pad_idx(eidx1, N_EDGES)])

    Xp = jnp.pad(X, ((0, N_NODES_PAD - N_NODES), (0, 0)))
    Wstk = jnp.stack([W00, W10])
    bstk = jnp.stack([b00, b10]).reshape(2, 1, FEAT)
    T1 = _tc_prep(Xp, Wstk, bstk)

    h1 = _sc_smooth(T1, NI, EI, dvp, dip)
    W2stk = jnp.stack([W01, W11])
    b2stk = jnp.stack([b01, b11]).reshape(2, 1, FEAT)
    T2 = _tc_mid(h1, W2stk, b2stk)
    h2 = _sc_smooth(T2, NI, EI, dvp, dip)

    out = _tc_final(h2, fc1_w, fc1_b, ln_g, ln_b, fc2_w, fc2_b)
    return out[0]


# speedup vs baseline: 7.8912x; 1.1514x over previous
import jax
import jax.numpy as jnp
from jax import lax
from jax.experimental import pallas as pl
from jax.experimental.pallas import tpu as pltpu
from jax.experimental.pallas import tpu_sc as plsc

N_NODES = 10000
N_NODES_PAD = 10240
N_EDGES = 5000
N_EDGES_PAD = 5120
N_MEM = 320000
N_MEM_PAD = 327680
NSUB = 16
FEAT = 64
NROW_T = N_NODES_PAD // NSUB
EROW_T = N_EDGES_PAD // NSUB
SUBROW = 320

CW = 16
CCH = 100
NCH = (N_MEM // NSUB) // CCH

SCCH = 128
SNCH = (N_MEM_PAD // NSUB) // SCCH
K = 4
NGRP = SNCH // K

R_BLK = 1024
NRB = N_NODES_PAD // R_BLK


def _counts_body(NI, EI, zn, ze, ones_h, outn, oute,
                 nbuf, ebuf, ones_v, accn, acce):
    c = lax.axis_index("c")
    s = lax.axis_index("s")
    pltpu.sync_copy(zn.at[pl.ds(s * NROW_T, NROW_T)],
                    accn.at[pl.ds(s * NROW_T, NROW_T)])
    pltpu.sync_copy(ze.at[pl.ds(s * EROW_T, EROW_T)],
                    acce.at[pl.ds(s * EROW_T, EROW_T)])
    pltpu.sync_copy(NI.at[c, pl.ds(s * NCH, NCH)], nbuf)
    pltpu.sync_copy(EI.at[c, pl.ds(s * NCH, NCH)], ebuf)
    pltpu.sync_copy(ones_h, ones_v)
    plsc.subcore_barrier()

    def body(j, carry):
        pltpu.sync_copy(ones_v, accn.at[nbuf.at[j]], add=True)
        pltpu.sync_copy(ones_v, acce.at[ebuf.at[j]], add=True)
        return carry

    lax.fori_loop(0, NCH, body, 0)
    plsc.subcore_barrier()
    pltpu.sync_copy(accn.at[pl.ds(s * NROW_T, NROW_T)],
                    outn.at[c, pl.ds(s * NROW_T, NROW_T)])
    pltpu.sync_copy(acce.at[pl.ds(s * EROW_T, EROW_T)],
                    oute.at[c, pl.ds(s * EROW_T, EROW_T)])


def _sc_counts(nidx0, eidx0, nidx1, eidx1):
    NI = jnp.stack([nidx0.reshape(-1, CCH), nidx1.reshape(-1, CCH)])
    EI = jnp.stack([eidx0.reshape(-1, CCH), eidx1.reshape(-1, CCH)])
    zn = jnp.zeros((N_NODES_PAD, CW), jnp.float32)
    ze = jnp.zeros((N_EDGES_PAD, CW), jnp.float32)
    ones_h = jnp.ones((CCH, CW), jnp.float32)
    mesh = plsc.VectorSubcoreMesh(core_axis_name="c", subcore_axis_name="s")
    fn = pl.kernel(
        _counts_body,
        mesh=mesh,
        compiler_params=pltpu.CompilerParams(use_tc_tiling_on_sc=False),
        out_type=[
            jax.ShapeDtypeStruct((2, N_NODES_PAD, CW), jnp.float32),
            jax.ShapeDtypeStruct((2, N_EDGES_PAD, CW), jnp.float32),
        ],
        scratch_types=[
            pltpu.VMEM((NCH, CCH), jnp.int32),
            pltpu.VMEM((NCH, CCH), jnp.int32),
            pltpu.VMEM((CCH, CW), jnp.float32),
            pltpu.VMEM_SHARED((N_NODES_PAD, CW), jnp.float32),
            pltpu.VMEM_SHARED((N_EDGES_PAD, CW), jnp.float32),
        ],
    )
    return fn(NI, EI, zn, ze, ones_h)


def _smooth_body(T, NI, EI, dvp, dip, zE, zS, O,
                 rows, di_v, dv_v, buf, TS_sh, E_sh):
    c = lax.axis_index("c")
    s = lax.axis_index("s")
    pltpu.sync_copy(dip.at[c, pl.ds(s * EROW_T, EROW_T)], di_v)
    pltpu.sync_copy(dvp.at[c, pl.ds(s * NROW_T, NROW_T)], dv_v)
    pltpu.sync_copy(zE.at[pl.ds(s * EROW_T, EROW_T)],
                    E_sh.at[pl.ds(s * EROW_T, EROW_T)])

    for half in range(2):
        base = s * NROW_T + half * SUBROW
        pltpu.sync_copy(T.at[c, pl.ds(base, SUBROW)], buf)

        def abody(r, carry):
            idx = jnp.zeros((16,), jnp.int32) + (half * SUBROW + r)
            d = plsc.load_gather(dv_v, [idx])
            for q in range(4):
                a = buf[r, pl.ds(q * 16, 16)]
                buf[r, pl.ds(q * 16, 16)] = a * d
            return carry

        lax.fori_loop(0, SUBROW, abody, 0)
        pltpu.sync_copy(buf, TS_sh.at[pl.ds(base, SUBROW)])
    plsc.subcore_barrier()

    def pairs_pass(IDXG, IDXS, src, dst):
        def scoped(gbuf, sbuf, semg):
            def body(g, carry):
                base = s * SNCH + g * K
                pltpu.sync_copy(IDXG.at[c, pl.ds(base, K)], gbuf)
                pltpu.sync_copy(IDXS.at[c, pl.ds(base, K)], sbuf)
                hs = [pltpu.async_copy(src.at[gbuf.at[b]],
                                       rows.at[pl.ds(b * SCCH, SCCH)], semg)
                      for b in range(K)]
                for b in range(K):
                    hs[b].wait()
                    pltpu.sync_copy(rows.at[pl.ds(b * SCCH, SCCH)],
                                    dst.at[sbuf.at[b]], add=True)
                return carry

            lax.fori_loop(0, NGRP, body, 0)

        pl.run_scoped(scoped,
                      pltpu.VMEM((K, SCCH), jnp.int32),
                      pltpu.VMEM((K, SCCH), jnp.int32),
                      pltpu.SemaphoreType.DMA)

    pairs_pass(NI, EI, TS_sh, E_sh)
    plsc.subcore_barrier()

    pltpu.sync_copy(E_sh.at[pl.ds(s * EROW_T, EROW_T)],
                    buf.at[pl.ds(0, EROW_T)])

    def cbody(r, carry):
        idx = jnp.zeros((16,), jnp.int32) + r
        d = plsc.load_gather(di_v, [idx])
        for q in range(4):
            a = buf[r, pl.ds(q * 16, 16)]
            buf[r, pl.ds(q * 16, 16)] = a * d
        return carry

    lax.fori_loop(0, EROW_T, cbody, 0)
    pltpu.sync_copy(buf.at[pl.ds(0, EROW_T)],
                    E_sh.at[pl.ds(s * EROW_T, EROW_T)])

    for half in range(2):
        base = s * NROW_T + half * SUBROW
        pltpu.sync_copy(zS.at[pl.ds(base, SUBROW)],
                        TS_sh.at[pl.ds(base, SUBROW)])
    plsc.subcore_barrier()

    pairs_pass(EI, NI, E_sh, TS_sh)
    plsc.subcore_barrier()

    for half in range(2):
        base = s * NROW_T + half * SUBROW
        pltpu.sync_copy(TS_sh.at[pl.ds(base, SUBROW)], buf)

        def fbody(r, carry):
            idx = jnp.zeros((16,), jnp.int32) + (half * SUBROW + r)
            d = plsc.load_gather(dv_v, [idx])
            for q in range(4):
                a = buf[r, pl.ds(q * 16, 16)]
                buf[r, pl.ds(q * 16, 16)] = jnp.maximum(a * d, 0.0)
            return carry

        lax.fori_loop(0, SUBROW, fbody, 0)
        pltpu.sync_copy(buf, O.at[c, pl.ds(base, SUBROW)])


def _sc_smooth(T, NI, EI, dvp, dip):
    zE = jnp.zeros((N_EDGES_PAD, FEAT), jnp.float32)
    zS = jnp.zeros((N_NODES_PAD, FEAT), jnp.float32)
    mesh = plsc.VectorSubcoreMesh(core_axis_name="c", subcore_axis_name="s")
    fn = pl.kernel(
        _smooth_body,
        mesh=mesh,
        compiler_params=pltpu.CompilerParams(needs_layout_passes=False,
                                             use_tc_tiling_on_sc=False),
        out_type=jax.ShapeDtypeStruct((2, N_NODES_PAD, FEAT), jnp.float32),
        scratch_types=[
            pltpu.VMEM((K * SCCH, FEAT), jnp.float32),
            pltpu.VMEM((EROW_T,), jnp.float32),
            pltpu.VMEM((NROW_T,), jnp.float32),
            pltpu.VMEM((SUBROW, FEAT), jnp.float32),
            pltpu.VMEM_SHARED((N_NODES_PAD, FEAT), jnp.float32),
            pltpu.VMEM_SHARED((N_EDGES_PAD, FEAT), jnp.float32),
        ],
    )
    return fn(T, NI, EI, dvp, dip, zE, zS)


def _prep_body(x_ref, w_ref, b_ref, o_ref):
    x = x_ref[...]
    w = w_ref[0]
    bias = b_ref[0, 0]
    o_ref[0] = jnp.dot(x, w, preferred_element_type=jnp.float32) + bias[None, :]


def _tc_prep(Xp, Wstk, bstk):
    return pl.pallas_call(
        _prep_body,
        grid=(2, NRB),
        in_specs=[
            pl.BlockSpec((R_BLK, 128), lambda b, r: (r, 0)),
            pl.BlockSpec((1, 128, FEAT), lambda b, r: (b, 0, 0)),
            pl.BlockSpec((1, 1, FEAT), lambda b, r: (b, 0, 0)),
        ],
        out_specs=pl.BlockSpec((1, R_BLK, FEAT), lambda b, r: (b, r, 0)),
        out_shape=jax.ShapeDtypeStruct((2, N_NODES_PAD, FEAT), jnp.float32),
    )(Xp, Wstk, bstk)


def _mid_body(o_ref, w_ref, b_ref, t_ref):
    o = o_ref[0]
    w = w_ref[0]
    bias = b_ref[0, 0]
    t_ref[0] = (jnp.dot(o, w, preferred_element_type=jnp.float32)
                + bias[None, :])


def _tc_mid(O_in, Wstk, bstk):
    return pl.pallas_call(
        _mid_body,
        grid=(2, NRB),
        in_specs=[
            pl.BlockSpec((1, R_BLK, FEAT), lambda b, r: (b, r, 0)),
            pl.BlockSpec((1, FEAT, FEAT), lambda b, r: (b, 0, 0)),
            pl.BlockSpec((1, 1, FEAT), lambda b, r: (b, 0, 0)),
        ],
        out_specs=pl.BlockSpec((1, R_BLK, FEAT), lambda b, r: (b, r, 0)),
        out_shape=jax.ShapeDtypeStruct((2, N_NODES_PAD, FEAT), jnp.float32),
    )(O_in, Wstk, bstk)


def _final_body(o_ref, fc1_w_ref, fc1_b_ref, ln_g_ref, ln_b_ref,
                fc2_w_ref, fc2_b_ref, out_ref):
    scale = 1.0 / N_NODES
    parts = [
        jnp.sum(o_ref[0], axis=0) * scale,
        jnp.sum(o_ref[1], axis=0) * scale,
    ]
    z = jnp.concatenate(parts, axis=-1)
    z = z @ fc1_w_ref[...] + fc1_b_ref[...]
    mu = jnp.mean(z)
    var = jnp.mean((z - mu) ** 2)
    z = (z - mu) / jnp.sqrt(var + 1e-5) * ln_g_ref[...] + ln_b_ref[...]
    z = z * jax.nn.sigmoid(z)
    z = z @ fc2_w_ref[...] + fc2_b_ref[...]
    nrm = jnp.maximum(jnp.sqrt(jnp.sum(z * z)), 1e-12)
    out_ref[0, :] = z / nrm


def _tc_final(O_l2, fc1_w, fc1_b, ln_g, ln_b, fc2_w, fc2_b):
    return pl.pallas_call(
        _final_body,
        out_shape=jax.ShapeDtypeStruct((1, 128), jnp.float32),
    )(O_l2, fc1_w, fc1_b, ln_g, ln_b, fc2_w, fc2_b)


def _pad_idx(idx, fill):
    return jnp.pad(idx, (0, N_MEM_PAD - N_MEM),
                   constant_values=fill).reshape(-1, SCCH)


def kernel(X, nidx0, eidx0, nidx1, eidx1, W00, b00, W01, b01, W10, b10, W11, b11, fc1_w, fc1_b, ln_g, ln_b, fc2_w, fc2_b):
    nidx0 = nidx0.astype(jnp.int32)
    eidx0 = eidx0.astype(jnp.int32)
    nidx1 = nidx1.astype(jnp.int32)
    eidx1 = eidx1.astype(jnp.int32)
    outn, oute = _sc_counts(nidx0, eidx0, nidx1, eidx1)
    dv = outn[:, :N_NODES, 0]
    de = oute[:, :N_EDGES, 0]
    dvis = jnp.where(dv > 0, lax.rsqrt(dv), 0.0)
    deinv = jnp.where(de > 0, 1.0 / de, 0.0)
    dvp = jnp.pad(dvis, ((0, 0), (0, N_NODES_PAD - N_NODES)))
    dip = jnp.pad(deinv, ((0, 0), (0, N_EDGES_PAD - N_EDGES)))

    NI = jnp.stack([_pad_idx(nidx0, N_NODES), _pad_idx(nidx1, N_NODES)])
    EI = jnp.stack([_pad_idx(eidx0, N_EDGES), _pad_idx(eidx1, N_EDGES)])

    Xp = jnp.pad(X, ((0, N_NODES_PAD - N_NODES), (0, 0)))
    Wstk = jnp.stack([W00, W10])
    bstk = jnp.stack([b00, b10]).reshape(2, 1, FEAT)
    T1 = _tc_prep(Xp, Wstk, bstk)

    h1 = _sc_smooth(T1, NI, EI, dvp, dip)
    W2stk = jnp.stack([W01, W11])
    b2stk = jnp.stack([b01, b11]).reshape(2, 1, FEAT)
    T2 = _tc_mid(h1, W2stk, b2stk)
    h2 = _sc_smooth(T2, NI, EI, dvp, dip)

    out = _tc_final(h2, fc1_w, fc1_b, ln_g, ln_b, fc2_w, fc2_b)
    return out[0]


# speedup vs baseline: 7.9414x; 1.0064x over previous
import jax
import jax.numpy as jnp
from jax import lax
from jax.experimental import pallas as pl
from jax.experimental.pallas import tpu as pltpu
from jax.experimental.pallas import tpu_sc as plsc

N_NODES = 10000
N_NODES_PAD = 10240
N_EDGES = 5000
N_EDGES_PAD = 5120
N_MEM = 320000
N_MEM_PAD = 327680
NSUB = 16
FEAT = 64
NROW_T = N_NODES_PAD // NSUB
EROW_T = N_EDGES_PAD // NSUB
SUBROW = 320

CW = 16
CCH = 100
NCH = (N_MEM // NSUB) // CCH

SCCH = 128
SNCH = (N_MEM_PAD // NSUB) // SCCH
K = 4
NGRP = SNCH // K

R_BLK = 1024
NRB = N_NODES_PAD // R_BLK


def _counts_body(NI, EI, zn, ze, ones_h, outn, oute,
                 nbuf, ebuf, ones_v, accn, acce):
    c = lax.axis_index("c")
    s = lax.axis_index("s")
    pltpu.sync_copy(zn.at[pl.ds(s * NROW_T, NROW_T)],
                    accn.at[pl.ds(s * NROW_T, NROW_T)])
    pltpu.sync_copy(ze.at[pl.ds(s * EROW_T, EROW_T)],
                    acce.at[pl.ds(s * EROW_T, EROW_T)])
    pltpu.sync_copy(NI.at[c, pl.ds(s * NCH, NCH)], nbuf)
    pltpu.sync_copy(EI.at[c, pl.ds(s * NCH, NCH)], ebuf)
    pltpu.sync_copy(ones_h, ones_v)
    plsc.subcore_barrier()

    def body(j, carry):
        pltpu.sync_copy(ones_v, accn.at[nbuf.at[j]], add=True)
        pltpu.sync_copy(ones_v, acce.at[ebuf.at[j]], add=True)
        return carry

    lax.fori_loop(0, NCH, body, 0)
    plsc.subcore_barrier()
    pltpu.sync_copy(accn.at[pl.ds(s * NROW_T, NROW_T)],
                    outn.at[c, pl.ds(s * NROW_T, NROW_T)])
    pltpu.sync_copy(acce.at[pl.ds(s * EROW_T, EROW_T)],
                    oute.at[c, pl.ds(s * EROW_T, EROW_T)])


def _sc_counts(nidx0, eidx0, nidx1, eidx1):
    NI = jnp.stack([nidx0.reshape(-1, CCH), nidx1.reshape(-1, CCH)])
    EI = jnp.stack([eidx0.reshape(-1, CCH), eidx1.reshape(-1, CCH)])
    zn = jnp.zeros((N_NODES_PAD, CW), jnp.float32)
    ze = jnp.zeros((N_EDGES_PAD, CW), jnp.float32)
    ones_h = jnp.ones((CCH, CW), jnp.float32)
    mesh = plsc.VectorSubcoreMesh(core_axis_name="c", subcore_axis_name="s")
    fn = pl.kernel(
        _counts_body,
        mesh=mesh,
        compiler_params=pltpu.CompilerParams(use_tc_tiling_on_sc=False),
        out_type=[
            jax.ShapeDtypeStruct((2, N_NODES_PAD, CW), jnp.float32),
            jax.ShapeDtypeStruct((2, N_EDGES_PAD, CW), jnp.float32),
        ],
        scratch_types=[
            pltpu.VMEM((NCH, CCH), jnp.int32),
            pltpu.VMEM((NCH, CCH), jnp.int32),
            pltpu.VMEM((CCH, CW), jnp.float32),
            pltpu.VMEM_SHARED((N_NODES_PAD, CW), jnp.float32),
            pltpu.VMEM_SHARED((N_EDGES_PAD, CW), jnp.float32),
        ],
    )
    return fn(NI, EI, zn, ze, ones_h)


def _smooth_body(T, NI, EI, dvp, dip, zE, zS, O,
                 rows, di_v, dv_v, buf, TS_sh, E_sh):
    c = lax.axis_index("c")
    s = lax.axis_index("s")
    pltpu.sync_copy(dip.at[c, pl.ds(s * EROW_T, EROW_T)], di_v)
    pltpu.sync_copy(dvp.at[c, pl.ds(s * NROW_T, NROW_T)], dv_v)
    pltpu.sync_copy(zE.at[pl.ds(s * EROW_T, EROW_T)],
                    E_sh.at[pl.ds(s * EROW_T, EROW_T)])

    for half in range(2):
        base = s * NROW_T + half * SUBROW
        pltpu.sync_copy(T.at[c, pl.ds(base, SUBROW)], buf)

        def abody(r, carry):
            idx = jnp.zeros((16,), jnp.int32) + (half * SUBROW + r)
            d = plsc.load_gather(dv_v, [idx])
            for q in range(4):
                a = buf[r, pl.ds(q * 16, 16)]
                buf[r, pl.ds(q * 16, 16)] = a * d
            return carry

        lax.fori_loop(0, SUBROW, abody, 0)
        pltpu.sync_copy(buf, TS_sh.at[pl.ds(base, SUBROW)])
    plsc.subcore_barrier()

    def pairs_pass(IDXG, IDXS, src, dst):
        def scoped(gbuf, sbuf, semg, sems):
            def body(g, carry):
                base = s * SNCH + g * K
                pltpu.sync_copy(IDXG.at[c, pl.ds(base, K)], gbuf)
                pltpu.sync_copy(IDXS.at[c, pl.ds(base, K)], sbuf)
                hs = [pltpu.async_copy(src.at[gbuf.at[b]],
                                       rows.at[pl.ds(b * SCCH, SCCH)], semg)
                      for b in range(K)]
                ss = []
                for b in range(K):
                    hs[b].wait()
                    ss.append(pltpu.async_copy(rows.at[pl.ds(b * SCCH, SCCH)],
                                               dst.at[sbuf.at[b]], sems,
                                               add=True))
                for h in ss:
                    h.wait()
                return carry

            lax.fori_loop(0, NGRP, body, 0)

        pl.run_scoped(scoped,
                      pltpu.VMEM((K, SCCH), jnp.int32),
                      pltpu.VMEM((K, SCCH), jnp.int32),
                      pltpu.SemaphoreType.DMA,
                      pltpu.SemaphoreType.DMA)

    pairs_pass(NI, EI, TS_sh, E_sh)
    plsc.subcore_barrier()

    pltpu.sync_copy(E_sh.at[pl.ds(s * EROW_T, EROW_T)],
                    buf.at[pl.ds(0, EROW_T)])

    def cbody(r, carry):
        idx = jnp.zeros((16,), jnp.int32) + r
        d = plsc.load_gather(di_v, [idx])
        for q in range(4):
            a = buf[r, pl.ds(q * 16, 16)]
            buf[r, pl.ds(q * 16, 16)] = a * d
        return carry

    lax.fori_loop(0, EROW_T, cbody, 0)
    pltpu.sync_copy(buf.at[pl.ds(0, EROW_T)],
                    E_sh.at[pl.ds(s * EROW_T, EROW_T)])

    for half in range(2):
        base = s * NROW_T + half * SUBROW
        pltpu.sync_copy(zS.at[pl.ds(base, SUBROW)],
                        TS_sh.at[pl.ds(base, SUBROW)])
    plsc.subcore_barrier()

    pairs_pass(EI, NI, E_sh, TS_sh)
    plsc.subcore_barrier()

    for half in range(2):
        base = s * NROW_T + half * SUBROW
        pltpu.sync_copy(TS_sh.at[pl.ds(base, SUBROW)], buf)

        def fbody(r, carry):
            idx = jnp.zeros((16,), jnp.int32) + (half * SUBROW + r)
            d = plsc.load_gather(dv_v, [idx])
            for q in range(4):
                a = buf[r, pl.ds(q * 16, 16)]
                buf[r, pl.ds(q * 16, 16)] = jnp.maximum(a * d, 0.0)
            return carry

        lax.fori_loop(0, SUBROW, fbody, 0)
        pltpu.sync_copy(buf, O.at[c, pl.ds(base, SUBROW)])


def _sc_smooth(T, NI, EI, dvp, dip):
    zE = jnp.zeros((N_EDGES_PAD, FEAT), jnp.float32)
    zS = jnp.zeros((N_NODES_PAD, FEAT), jnp.float32)
    mesh = plsc.VectorSubcoreMesh(core_axis_name="c", subcore_axis_name="s")
    fn = pl.kernel(
        _smooth_body,
        mesh=mesh,
        compiler_params=pltpu.CompilerParams(needs_layout_passes=False,
                                             use_tc_tiling_on_sc=False),
        out_type=jax.ShapeDtypeStruct((2, N_NODES_PAD, FEAT), jnp.float32),
        scratch_types=[
            pltpu.VMEM((K * SCCH, FEAT), jnp.float32),
            pltpu.VMEM((EROW_T,), jnp.float32),
            pltpu.VMEM((NROW_T,), jnp.float32),
            pltpu.VMEM((SUBROW, FEAT), jnp.float32),
            pltpu.VMEM_SHARED((N_NODES_PAD, FEAT), jnp.float32),
            pltpu.VMEM_SHARED((N_EDGES_PAD, FEAT), jnp.float32),
        ],
    )
    return fn(T, NI, EI, dvp, dip, zE, zS)


def _prep_body(x_ref, w_ref, b_ref, o_ref):
    x = x_ref[...]
    w = w_ref[0]
    bias = b_ref[0, 0]
    o_ref[0] = jnp.dot(x, w, preferred_element_type=jnp.float32) + bias[None, :]


def _tc_prep(Xp, Wstk, bstk):
    return pl.pallas_call(
        _prep_body,
        grid=(2, NRB),
        in_specs=[
            pl.BlockSpec((R_BLK, 128), lambda b, r: (r, 0)),
            pl.BlockSpec((1, 128, FEAT), lambda b, r: (b, 0, 0)),
            pl.BlockSpec((1, 1, FEAT), lambda b, r: (b, 0, 0)),
        ],
        out_specs=pl.BlockSpec((1, R_BLK, FEAT), lambda b, r: (b, r, 0)),
        out_shape=jax.ShapeDtypeStruct((2, N_NODES_PAD, FEAT), jnp.float32),
    )(Xp, Wstk, bstk)


def _mid_body(o_ref, w_ref, b_ref, t_ref):
    o = o_ref[0]
    w = w_ref[0]
    bias = b_ref[0, 0]
    t_ref[0] = (jnp.dot(o, w, preferred_element_type=jnp.float32)
                + bias[None, :])


def _tc_mid(O_in, Wstk, bstk):
    return pl.pallas_call(
        _mid_body,
        grid=(2, NRB),
        in_specs=[
            pl.BlockSpec((1, R_BLK, FEAT), lambda b, r: (b, r, 0)),
            pl.BlockSpec((1, FEAT, FEAT), lambda b, r: (b, 0, 0)),
            pl.BlockSpec((1, 1, FEAT), lambda b, r: (b, 0, 0)),
        ],
        out_specs=pl.BlockSpec((1, R_BLK, FEAT), lambda b, r: (b, r, 0)),
        out_shape=jax.ShapeDtypeStruct((2, N_NODES_PAD, FEAT), jnp.float32),
    )(O_in, Wstk, bstk)


def _final_body(o_ref, fc1_w_ref, fc1_b_ref, ln_g_ref, ln_b_ref,
                fc2_w_ref, fc2_b_ref, out_ref):
    scale = 1.0 / N_NODES
    parts = [
        jnp.sum(o_ref[0], axis=0) * scale,
        jnp.sum(o_ref[1], axis=0) * scale,
    ]
    z = jnp.concatenate(parts, axis=-1)
    z = z @ fc1_w_ref[...] + fc1_b_ref[...]
    mu = jnp.mean(z)
    var = jnp.mean((z - mu) ** 2)
    z = (z - mu) / jnp.sqrt(var + 1e-5) * ln_g_ref[...] + ln_b_ref[...]
    z = z * jax.nn.sigmoid(z)
    z = z @ fc2_w_ref[...] + fc2_b_ref[...]
    nrm = jnp.maximum(jnp.sqrt(jnp.sum(z * z)), 1e-12)
    out_ref[0, :] = z / nrm


def _tc_final(O_l2, fc1_w, fc1_b, ln_g, ln_b, fc2_w, fc2_b):
    return pl.pallas_call(
        _final_body,
        out_shape=jax.ShapeDtypeStruct((1, 128), jnp.float32),
    )(O_l2, fc1_w, fc1_b, ln_g, ln_b, fc2_w, fc2_b)


def _pad_idx(idx, fill):
    return jnp.pad(idx, (0, N_MEM_PAD - N_MEM),
                   constant_values=fill).reshape(-1, SCCH)


def kernel(X, nidx0, eidx0, nidx1, eidx1, W00, b00, W01, b01, W10, b10, W11, b11, fc1_w, fc1_b, ln_g, ln_b, fc2_w, fc2_b):
    nidx0 = nidx0.astype(jnp.int32)
    eidx0 = eidx0.astype(jnp.int32)
    nidx1 = nidx1.astype(jnp.int32)
    eidx1 = eidx1.astype(jnp.int32)
    outn, oute = _sc_counts(nidx0, eidx0, nidx1, eidx1)
    dv = outn[:, :N_NODES, 0]
    de = oute[:, :N_EDGES, 0]
    dvis = jnp.where(dv > 0, lax.rsqrt(dv), 0.0)
    deinv = jnp.where(de > 0, 1.0 / de, 0.0)
    dvp = jnp.pad(dvis, ((0, 0), (0, N_NODES_PAD - N_NODES)))
    dip = jnp.pad(deinv, ((0, 0), (0, N_EDGES_PAD - N_EDGES)))

    NI = jnp.stack([_pad_idx(nidx0, N_NODES), _pad_idx(nidx1, N_NODES)])
    EI = jnp.stack([_pad_idx(eidx0, N_EDGES), _pad_idx(eidx1, N_EDGES)])

    Xp = jnp.pad(X, ((0, N_NODES_PAD - N_NODES), (0, 0)))
    Wstk = jnp.stack([W00, W10])
    bstk = jnp.stack([b00, b10]).reshape(2, 1, FEAT)
    T1 = _tc_prep(Xp, Wstk, bstk)

    h1 = _sc_smooth(T1, NI, EI, dvp, dip)
    W2stk = jnp.stack([W01, W11])
    b2stk = jnp.stack([b01, b11]).reshape(2, 1, FEAT)
    T2 = _tc_mid(h1, W2stk, b2stk)
    h2 = _sc_smooth(T2, NI, EI, dvp, dip)

    out = _tc_final(h2, fc1_w, fc1_b, ln_g, ln_b, fc2_w, fc2_b)
    return out[0]
